# Initial kernel scaffold; baseline (speedup 1.0000x reference)
import jax
import jax.numpy as jnp
from jax.experimental import pallas as pl


def kernel(x, weight):
    raise NotImplementedError("write your pallas kernel here")


# speedup vs baseline: 1.4941x; 1.4941x over previous
import functools

import jax
import jax.numpy as jnp
from jax import lax
from jax.experimental import pallas as pl
from jax.experimental.pallas import tpu as pltpu
from jax.experimental.pallas import tpu_sc as plsc

NC = 2
NS = 16
NW = NC * NS

CH = 128


def _gather_call(x2d, weight):
    n_rows, ch = x2d.shape
    assert ch == CH and n_rows % NW == 0
    nch = n_rows // NW
    bpw = nch * CH
    B = n_rows * CH
    D = weight.shape[1]

    mesh = plsc.VectorSubcoreMesh(core_axis_name="c", subcore_axis_name="s")

    @functools.partial(
        pl.kernel,
        mesh=mesh,
        out_type=jax.ShapeDtypeStruct((B, D), jnp.float32),
        scratch_types=[
            pltpu.VMEM((nch, CH), jnp.int32),
            pltpu.VMEM((CH, D), jnp.float32),
            pltpu.VMEM((CH, D), jnp.float32),
            pltpu.SemaphoreType.DMA,
            pltpu.SemaphoreType.DMA,
        ],
    )
    def k(idx_hbm, table_hbm, out_hbm, idx_v, rows_a, rows_b, sem_a, sem_b):
        wid = lax.axis_index("s") * NC + lax.axis_index("c")
        base = wid * bpw
        pltpu.sync_copy(idx_hbm.at[pl.ds(wid * nch, nch)], idx_v)

        bufs = (rows_a, rows_b)
        sems = (sem_a, sem_b)
        copies = [None] * nch
        copies[0] = pltpu.async_copy(table_hbm.at[idx_v.at[0]], bufs[0], sems[0])
        for j in range(nch):
            if j + 1 < nch:
                copies[j + 1] = pltpu.async_copy(
                    table_hbm.at[idx_v.at[j + 1]], bufs[(j + 1) % 2],
                    sems[(j + 1) % 2])
            copies[j].wait()
            pltpu.sync_copy(bufs[j % 2], out_hbm.at[pl.ds(base + j * CH, CH)])

    return k(x2d, weight)


def kernel(x, weight):
    B = x.size
    x2d = x.reshape(B // CH, CH)
    out = _gather_call(x2d, weight)
    return out.reshape(x.shape + (weight.shape[1],))


# speedup vs baseline: 1.5279x; 1.0227x over previous
import functools

import jax
import jax.numpy as jnp
from jax import lax
from jax.experimental import pallas as pl
from jax.experimental.pallas import tpu as pltpu
from jax.experimental.pallas import tpu_sc as plsc

NC = 2
NS = 16
NW = NC * NS

CH = 128


def _gather_call(x2d, weight):
    n_rows, ch = x2d.shape
    assert ch == CH and n_rows % NW == 0
    nch = n_rows // NW
    bpw = nch * CH
    B = n_rows * CH
    D = weight.shape[1]

    mesh = plsc.VectorSubcoreMesh(core_axis_name="c", subcore_axis_name="s")
    NBUF = 4

    @functools.partial(
        pl.kernel,
        mesh=mesh,
        out_type=jax.ShapeDtypeStruct((B, D), jnp.float32),
        scratch_types=(
            [pltpu.VMEM((nch, CH), jnp.int32)]
            + [pltpu.VMEM((CH, D), jnp.float32) for _ in range(NBUF)]
            + [pltpu.SemaphoreType.DMA for _ in range(2 * NBUF)]
        ),
    )
    def k(idx_hbm, table_hbm, out_hbm, idx_v, *rest):
        bufs = rest[:NBUF]
        gsem = rest[NBUF:2 * NBUF]
        ssem = rest[2 * NBUF:]
        wid = lax.axis_index("s") * NC + lax.axis_index("c")
        base = wid * bpw
        pltpu.sync_copy(idx_hbm.at[pl.ds(wid * nch, nch)], idx_v)

        gathers = [None] * nch
        stores = [None] * nch
        for j in range(min(NBUF - 1, nch)):
            gathers[j] = pltpu.async_copy(
                table_hbm.at[idx_v.at[j]], bufs[j % NBUF], gsem[j % NBUF])
        for j in range(nch):
            gathers[j].wait()
            stores[j] = pltpu.async_copy(
                bufs[j % NBUF], out_hbm.at[pl.ds(base + j * CH, CH)],
                ssem[j % NBUF])
            nxt = j + NBUF - 1
            if nxt < nch:
                prev = nxt - NBUF
                if prev >= 0:
                    stores[prev].wait()
                    stores[prev] = None
                gathers[nxt] = pltpu.async_copy(
                    table_hbm.at[idx_v.at[nxt]], bufs[nxt % NBUF],
                    gsem[nxt % NBUF])
        for st in stores:
            if st is not None:
                st.wait()

    return k(x2d, weight)


def kernel(x, weight):
    B = x.size
    x2d = x.reshape(B // CH, CH)
    out = _gather_call(x2d, weight)
    return out.reshape(x.shape + (weight.shape[1],))


# speedup vs baseline: 1.5664x; 1.0252x over previous
import functools

import jax
import jax.numpy as jnp
from jax import lax
from jax.experimental import pallas as pl
from jax.experimental.pallas import tpu as pltpu
from jax.experimental.pallas import tpu_sc as plsc

NC = 2
NS = 16
NW = NC * NS

CH = 128


def kernel(x, weight):
    R, C = x.shape
    D = weight.shape[1]
    B = R * C
    bpw = B // NW
    nch = bpw // CH
    wpr = C // bpw
    assert bpw % CH == 0 and C % bpw == 0

    mesh = plsc.VectorSubcoreMesh(core_axis_name="c", subcore_axis_name="s")
    NBUF = 6

    @functools.partial(
        pl.kernel,
        mesh=mesh,
        out_type=jax.ShapeDtypeStruct((R, C, D), jnp.float32),
        scratch_types=(
            [pltpu.VMEM((bpw,), jnp.int32)]
            + [pltpu.VMEM((CH, D), jnp.float32) for _ in range(NBUF)]
            + [pltpu.SemaphoreType.DMA for _ in range(2 * NBUF)]
        ),
    )
    def k(idx_hbm, table_hbm, out_hbm, idx_v, *rest):
        bufs = rest[:NBUF]
        gsem = rest[NBUF:2 * NBUF]
        ssem = rest[2 * NBUF:]
        wid = lax.axis_index("s") * NC + lax.axis_index("c")
        row = wid // wpr
        col = (wid % wpr) * bpw
        pltpu.sync_copy(idx_hbm.at[row, pl.ds(col, bpw)], idx_v)

        gathers = [None] * nch
        stores = [None] * nch
        for j in range(min(NBUF - 1, nch)):
            gathers[j] = pltpu.async_copy(
                table_hbm.at[idx_v.at[pl.ds(j * CH, CH)]], bufs[j % NBUF],
                gsem[j % NBUF])
        for j in range(nch):
            gathers[j].wait()
            stores[j] = pltpu.async_copy(
                bufs[j % NBUF],
                out_hbm.at[row, pl.ds(col + j * CH, CH)],
                ssem[j % NBUF])
            nxt = j + NBUF - 1
            if nxt < nch:
                prev = nxt - NBUF
                if prev >= 0:
                    stores[prev].wait()
                    stores[prev] = None
                gathers[nxt] = pltpu.async_copy(
                    table_hbm.at[idx_v.at[pl.ds(nxt * CH, CH)]],
                    bufs[nxt % NBUF], gsem[nxt % NBUF])
        for st in stores:
            if st is not None:
                st.wait()

    return k(x, weight)
